# Initial kernel scaffold; baseline (speedup 1.0000x reference)
import jax
import jax.numpy as jnp
from jax.experimental import pallas as pl


def kernel(x, domain_ids, shared_table, prior_table, ln_gamma, ln_beta, W0, b0, W1, b1, dom_emb, Wd, bd, Wg, bg, T0, bt0, T1, bt1, T2, bt2):
    raise NotImplementedError("write your pallas kernel here")


# speedup vs baseline: 1.3645x; 1.3645x over previous
import functools

import jax
import jax.numpy as jnp
from jax import lax
from jax.experimental import pallas as pl
from jax.experimental.pallas import tpu as pltpu
from jax.experimental.pallas import tpu_sc as plsc

_B = 16384
_F = 26
_D = 16
_IN = _F * _D
_DOM = 4
_TOTAL = _B * _F

_NC = 2
_NS = 16
_NW = _NC * _NS
_PER_W = _TOTAL // _NW
_NCH = 8
_CH = _PER_W // _NCH

@functools.cache
def _make_sc_gather():
    mesh = plsc.VectorSubcoreMesh(core_axis_name="c", subcore_axis_name="s")

    @functools.partial(
        pl.kernel,
        out_type=jax.ShapeDtypeStruct((_TOTAL, _D), jnp.float32),
        mesh=mesh,
        scratch_types=[
            pltpu.VMEM((_PER_W,), jnp.int32),
            pltpu.VMEM((_CH, _D), jnp.float32),
            pltpu.VMEM((_CH, _D), jnp.float32),
            pltpu.SemaphoreType.DMA,
            pltpu.SemaphoreType.DMA,
            pltpu.SemaphoreType.DMA,
            pltpu.SemaphoreType.DMA,
        ],
        compiler_params=pltpu.CompilerParams(use_tc_tiling_on_sc=False),
    )
    def _sc_gather(table_hbm, idx_hbm, out_hbm, idx_v, rows0, rows1,
                   g0, g1, s0, s1):
        wid = lax.axis_index("s") * _NC + lax.axis_index("c")
        base = wid * _PER_W
        pltpu.sync_copy(idx_hbm.at[pl.ds(base, _PER_W)], idx_v)
        rows = (rows0, rows1)
        gsem = (g0, g1)
        ssem = (s0, s1)
        cps = [None, None]
        sps = [None, None]
        cps[0] = pltpu.async_copy(
            table_hbm.at[idx_v.at[pl.ds(0, _CH)]], rows[0], gsem[0])
        for c in range(_NCH):
            p = c & 1
            q = p ^ 1
            cps[p].wait()
            sps[p] = pltpu.async_copy(
                rows[p], out_hbm.at[pl.ds(base + c * _CH, _CH)], ssem[p])
            if c + 1 < _NCH:
                if c >= 1:
                    sps[q].wait()
                cps[q] = pltpu.async_copy(
                    table_hbm.at[idx_v.at[pl.ds((c + 1) * _CH, _CH)]],
                    rows[q], gsem[q])
        sps[(_NCH - 2) & 1].wait()
        sps[(_NCH - 1) & 1].wait()

    return _sc_gather


_BT = 1024
_GRID = _B // _BT


def _tc_body(emb_ref, dom_ref, lg_ref, lb_ref, W0_ref, b0_ref, W1_ref, b1_ref,
             de_ref, Wd_ref, bd_ref, Wg_ref, bg_ref, T0_ref, bt0_ref,
             T1_ref, bt1_ref, T2_ref, bt2_ref, out_ref):
    f32 = jnp.float32
    emb = emb_ref[...]
    mu = jnp.mean(emb, axis=-1, keepdims=True)
    var = jnp.mean((emb - mu) ** 2, axis=-1, keepdims=True)
    h = (emb - mu) * lax.rsqrt(var + 1e-5) * lg_ref[...] + lb_ref[...]
    h = jnp.maximum(jnp.dot(h, W0_ref[...], preferred_element_type=f32)
                    + b0_ref[...], 0.0)
    h = jnp.maximum(jnp.dot(h, W1_ref[...], preferred_element_type=f32)
                    + b1_ref[...], 0.0)
    graw = jnp.dot(de_ref[...], Wd_ref[...], preferred_element_type=f32) \
        + bd_ref[...]
    gl = jnp.dot(graw, Wg_ref[...], preferred_element_type=f32) + bg_ref[...]
    gate = graw * jax.nn.sigmoid(gl)
    dom = dom_ref[...]
    onehot = (dom[:, None] ==
              lax.broadcasted_iota(jnp.int32, (1, _DOM), 1)).astype(f32)
    pers = h * jnp.dot(onehot, gate, preferred_element_type=f32)
    t = jnp.maximum(jnp.dot(pers, T0_ref[...], preferred_element_type=f32)
                    + bt0_ref[...], 0.0)
    t = jnp.maximum(jnp.dot(t, T1_ref[...], preferred_element_type=f32)
                    + bt1_ref[...], 0.0)
    logits = jnp.sum(t * T2_ref[...], axis=-1) + bt2_ref[0, 0]
    out_ref[...] = jax.nn.sigmoid(logits)


def _full(shape):
    return pl.BlockSpec(shape, lambda i: tuple(0 for _ in shape))


_tc_forward = pl.pallas_call(
    _tc_body,
    grid=(_GRID,),
    in_specs=[
        pl.BlockSpec((_BT, _IN), lambda i: (i, 0)),
        pl.BlockSpec((_BT,), lambda i: (i,)),
        _full((1, _IN)),
        _full((1, _IN)),
        _full((_IN, 512)),
        _full((1, 512)),
        _full((512, 256)),
        _full((1, 256)),
        _full((_DOM, _D)),
        _full((_D, 256)),
        _full((1, 256)),
        _full((256, 256)),
        _full((1, 256)),
        _full((256, 128)),
        _full((1, 128)),
        _full((128, 64)),
        _full((1, 64)),
        _full((1, 64)),
        pl.BlockSpec(memory_space=pltpu.SMEM),
    ],
    out_specs=pl.BlockSpec((_BT,), lambda i: (i,)),
    out_shape=jax.ShapeDtypeStruct((_B,), jnp.float32),
    compiler_params=pltpu.CompilerParams(
        dimension_semantics=("parallel",)),
)


def kernel(x, domain_ids, shared_table, prior_table, ln_gamma, ln_beta,
           W0, b0, W1, b1, dom_emb, Wd, bd, Wg, bg,
           T0, bt0, T1, bt1, T2, bt2):
    idx = x.reshape(_TOTAL).astype(jnp.int32)
    rows = _make_sc_gather()(shared_table, idx)
    emb = rows.reshape(_B, _IN)
    return _tc_forward(
        emb, domain_ids.astype(jnp.int32),
        ln_gamma.reshape(1, _IN), ln_beta.reshape(1, _IN),
        W0, b0.reshape(1, 512), W1, b1.reshape(1, 256),
        dom_emb, Wd, bd.reshape(1, 256), Wg, bg.reshape(1, 256),
        T0, bt0.reshape(1, 128), T1, bt1.reshape(1, 64),
        T2.reshape(1, 64), bt2.reshape(1, 1))


# speedup vs baseline: 1.7249x; 1.2641x over previous
import functools

import jax
import jax.numpy as jnp
from jax import lax
from jax.experimental import pallas as pl
from jax.experimental.pallas import tpu as pltpu
from jax.experimental.pallas import tpu_sc as plsc

_B = 16384
_F = 26
_D = 16
_IN = _F * _D
_DOM = 4
_TOTAL = _B * _F

_VB = 8192
_V = 1000000
_RB = _VB * _D // 128


def _tr_body(in_ref, out_ref):
    x = in_ref[...]
    y = x.T
    y3 = y.reshape(_RB, 8, _D)
    out_ref[...] = jnp.concatenate([y3[:, j, :] for j in range(8)], axis=-1)


_tr_table = pl.pallas_call(
    _tr_body,
    grid=(-(-_V // _VB),),
    in_specs=[pl.BlockSpec((_D, _VB), lambda i: (0, i))],
    out_specs=pl.BlockSpec((_RB, 128), lambda i: (i, 0)),
    out_shape=jax.ShapeDtypeStruct((_V * _D // 128, 128), jnp.float32),
    compiler_params=pltpu.CompilerParams(
        dimension_semantics=("arbitrary",)),
)


_NC = 2
_NS = 16
_NW = _NC * _NS
_PER_W = _TOTAL // _NW
_NCH = 8
_CH = _PER_W // _NCH

@functools.cache
def _make_sc_gather():
    mesh = plsc.VectorSubcoreMesh(core_axis_name="c", subcore_axis_name="s")

    @functools.partial(
        pl.kernel,
        out_type=jax.ShapeDtypeStruct((_TOTAL, _D), jnp.float32),
        mesh=mesh,
        scratch_types=[
            pltpu.VMEM((_PER_W,), jnp.int32),
            pltpu.VMEM((_CH, _D), jnp.float32),
            pltpu.VMEM((_CH, _D), jnp.float32),
            pltpu.SemaphoreType.DMA,
            pltpu.SemaphoreType.DMA,
            pltpu.SemaphoreType.DMA,
            pltpu.SemaphoreType.DMA,
        ],
        compiler_params=pltpu.CompilerParams(use_tc_tiling_on_sc=False),
    )
    def _sc_gather(table_hbm, idx_hbm, out_hbm, idx_v, rows0, rows1,
                   g0, g1, s0, s1):
        wid = lax.axis_index("s") * _NC + lax.axis_index("c")
        base = wid * _PER_W
        pltpu.sync_copy(idx_hbm.at[pl.ds(base, _PER_W)], idx_v)
        rows = (rows0, rows1)
        gsem = (g0, g1)
        ssem = (s0, s1)
        cps = [None, None]
        sps = [None, None]
        cps[0] = pltpu.async_copy(
            table_hbm.at[idx_v.at[pl.ds(0, _CH)]], rows[0], gsem[0])
        for c in range(_NCH):
            p = c & 1
            q = p ^ 1
            cps[p].wait()
            sps[p] = pltpu.async_copy(
                rows[p], out_hbm.at[pl.ds(base + c * _CH, _CH)], ssem[p])
            if c + 1 < _NCH:
                if c >= 1:
                    sps[q].wait()
                cps[q] = pltpu.async_copy(
                    table_hbm.at[idx_v.at[pl.ds((c + 1) * _CH, _CH)]],
                    rows[q], gsem[q])
        sps[(_NCH - 2) & 1].wait()
        sps[(_NCH - 1) & 1].wait()

    return _sc_gather


_BT = 1024
_GRID = _B // _BT


def _tc_body(emb_ref, dom_ref, lg_ref, lb_ref, W0_ref, b0_ref, W1_ref, b1_ref,
             de_ref, Wd_ref, bd_ref, Wg_ref, bg_ref, T0_ref, bt0_ref,
             T1_ref, bt1_ref, T2_ref, bt2_ref, out_ref):
    f32 = jnp.float32
    emb = emb_ref[...]
    mu = jnp.mean(emb, axis=-1, keepdims=True)
    var = jnp.mean((emb - mu) ** 2, axis=-1, keepdims=True)
    h = (emb - mu) * lax.rsqrt(var + 1e-5) * lg_ref[...] + lb_ref[...]
    h = jnp.maximum(jnp.dot(h, W0_ref[...], preferred_element_type=f32)
                    + b0_ref[...], 0.0)
    h = jnp.maximum(jnp.dot(h, W1_ref[...], preferred_element_type=f32)
                    + b1_ref[...], 0.0)
    graw = jnp.dot(de_ref[...], Wd_ref[...], preferred_element_type=f32) \
        + bd_ref[...]
    gl = jnp.dot(graw, Wg_ref[...], preferred_element_type=f32) + bg_ref[...]
    gate = graw * jax.nn.sigmoid(gl)
    dom = dom_ref[...]
    onehot = (dom[:, None] ==
              lax.broadcasted_iota(jnp.int32, (1, _DOM), 1)).astype(f32)
    pers = h * jnp.dot(onehot, gate, preferred_element_type=f32)
    t = jnp.maximum(jnp.dot(pers, T0_ref[...], preferred_element_type=f32)
                    + bt0_ref[...], 0.0)
    t = jnp.maximum(jnp.dot(t, T1_ref[...], preferred_element_type=f32)
                    + bt1_ref[...], 0.0)
    logits = jnp.sum(t * T2_ref[...], axis=-1) + bt2_ref[0, 0]
    out_ref[...] = jax.nn.sigmoid(logits)


def _full(shape):
    return pl.BlockSpec(shape, lambda i: tuple(0 for _ in shape))


_tc_forward = pl.pallas_call(
    _tc_body,
    grid=(_GRID,),
    in_specs=[
        pl.BlockSpec((_BT, _IN), lambda i: (i, 0)),
        pl.BlockSpec((_BT,), lambda i: (i,)),
        _full((1, _IN)),
        _full((1, _IN)),
        _full((_IN, 512)),
        _full((1, 512)),
        _full((512, 256)),
        _full((1, 256)),
        _full((_DOM, _D)),
        _full((_D, 256)),
        _full((1, 256)),
        _full((256, 256)),
        _full((1, 256)),
        _full((256, 128)),
        _full((1, 128)),
        _full((128, 64)),
        _full((1, 64)),
        _full((1, 64)),
        pl.BlockSpec(memory_space=pltpu.SMEM),
    ],
    out_specs=pl.BlockSpec((_BT,), lambda i: (i,)),
    out_shape=jax.ShapeDtypeStruct((_B,), jnp.float32),
    compiler_params=pltpu.CompilerParams(
        dimension_semantics=("parallel",)),
)


def kernel(x, domain_ids, shared_table, prior_table, ln_gamma, ln_beta,
           W0, b0, W1, b1, dom_emb, Wd, bd, Wg, bg,
           T0, bt0, T1, bt1, T2, bt2):
    idx = x.reshape(_TOTAL).astype(jnp.int32)
    table_lin = _tr_table(shared_table.T).reshape(_V, _D)
    rows = _make_sc_gather()(table_lin, idx)
    emb = rows.reshape(_B, _IN)
    return _tc_forward(
        emb, domain_ids.astype(jnp.int32),
        ln_gamma.reshape(1, _IN), ln_beta.reshape(1, _IN),
        W0, b0.reshape(1, 512), W1, b1.reshape(1, 256),
        dom_emb, Wd, bd.reshape(1, 256), Wg, bg.reshape(1, 256),
        T0, bt0.reshape(1, 128), T1, bt1.reshape(1, 64),
        T2.reshape(1, 64), bt2.reshape(1, 1))


# speedup vs baseline: 4.0508x; 2.3484x over previous
import functools

import jax
import jax.numpy as jnp
from jax import lax
from jax.experimental import pallas as pl
from jax.experimental.pallas import tpu as pltpu
from jax.experimental.pallas import tpu_sc as plsc

_B = 16384
_F = 26
_D = 16
_IN = _F * _D
_DOM = 4
_TOTAL = _B * _F

_VB = 65536
_V = 1000000
_TR_GRID = -(-_V // _VB)
_V_PAD = _TR_GRID * _VB
_RB = _VB * _D // 128


def _tr_body(in_ref, out_ref):
    x = in_ref[...]
    for g8 in range(_VB // 1024):
        w = jnp.concatenate(
            [x[:, (g8 * 8 + k) * 128:(g8 * 8 + k + 1) * 128]
             for k in range(8)], axis=0)
        out_ref[pl.ds(g8 * 128, 128), :] = w.T


_tr_table = pl.pallas_call(
    _tr_body,
    grid=(_TR_GRID,),
    in_specs=[pl.BlockSpec((_D, _VB), lambda i: (0, i))],
    out_specs=pl.BlockSpec((_RB, 128), lambda i: (i, 0)),
    out_shape=jax.ShapeDtypeStruct((_V_PAD * _D // 128, 128), jnp.float32),
    compiler_params=pltpu.CompilerParams(
        dimension_semantics=("arbitrary",)),
)


_VB_BITS = _VB.bit_length() - 1


def _permute_idx(idx):
    blk = idx >> _VB_BITS
    vb = idx & (_VB - 1)
    g8 = vb >> 10
    k = (vb >> 7) & 7
    c = vb & 127
    return (blk << _VB_BITS) + (g8 << 10) + (c << 3) + k


_NC = 2
_NS = 16
_NW = _NC * _NS
_PER_W = _TOTAL // _NW
_NCH = 8
_CH = _PER_W // _NCH

@functools.cache
def _make_sc_gather():
    mesh = plsc.VectorSubcoreMesh(core_axis_name="c", subcore_axis_name="s")

    @functools.partial(
        pl.kernel,
        out_type=jax.ShapeDtypeStruct((_TOTAL, _D), jnp.float32),
        mesh=mesh,
        scratch_types=[
            pltpu.VMEM((_PER_W,), jnp.int32),
            pltpu.VMEM((_CH, _D), jnp.float32),
            pltpu.VMEM((_CH, _D), jnp.float32),
            pltpu.SemaphoreType.DMA,
            pltpu.SemaphoreType.DMA,
            pltpu.SemaphoreType.DMA,
            pltpu.SemaphoreType.DMA,
        ],
        compiler_params=pltpu.CompilerParams(use_tc_tiling_on_sc=False),
    )
    def _sc_gather(table_hbm, idx_hbm, out_hbm, idx_v, rows0, rows1,
                   g0, g1, s0, s1):
        wid = lax.axis_index("s") * _NC + lax.axis_index("c")
        base = wid * _PER_W
        pltpu.sync_copy(idx_hbm.at[pl.ds(base, _PER_W)], idx_v)
        rows = (rows0, rows1)
        gsem = (g0, g1)
        ssem = (s0, s1)
        cps = [None, None]
        sps = [None, None]
        cps[0] = pltpu.async_copy(
            table_hbm.at[idx_v.at[pl.ds(0, _CH)]], rows[0], gsem[0])
        for c in range(_NCH):
            p = c & 1
            q = p ^ 1
            cps[p].wait()
            sps[p] = pltpu.async_copy(
                rows[p], out_hbm.at[pl.ds(base + c * _CH, _CH)], ssem[p])
            if c + 1 < _NCH:
                if c >= 1:
                    sps[q].wait()
                cps[q] = pltpu.async_copy(
                    table_hbm.at[idx_v.at[pl.ds((c + 1) * _CH, _CH)]],
                    rows[q], gsem[q])
        sps[(_NCH - 2) & 1].wait()
        sps[(_NCH - 1) & 1].wait()

    return _sc_gather


_BT = 2048
_GRID = _B // _BT


def _tc_body(emb_ref, dom_ref, lg_ref, lb_ref, W0_ref, b0_ref, W1_ref, b1_ref,
             de_ref, Wd_ref, bd_ref, Wg_ref, bg_ref, T0_ref, bt0_ref,
             T1_ref, bt1_ref, T2_ref, bt2_ref, out_ref):
    f32 = jnp.float32
    emb = emb_ref[...]
    mu = jnp.mean(emb, axis=-1, keepdims=True)
    var = jnp.mean((emb - mu) ** 2, axis=-1, keepdims=True)
    h = (emb - mu) * lax.rsqrt(var + 1e-5) * lg_ref[...] + lb_ref[...]
    h = jnp.maximum(jnp.dot(h, W0_ref[...], preferred_element_type=f32)
                    + b0_ref[...], 0.0)
    h = jnp.maximum(jnp.dot(h, W1_ref[...], preferred_element_type=f32)
                    + b1_ref[...], 0.0)
    graw = jnp.dot(de_ref[...], Wd_ref[...], preferred_element_type=f32) \
        + bd_ref[...]
    gl = jnp.dot(graw, Wg_ref[...], preferred_element_type=f32) + bg_ref[...]
    gate = graw * jax.nn.sigmoid(gl)
    dom = dom_ref[...]
    onehot = (dom[:, None] ==
              lax.broadcasted_iota(jnp.int32, (1, _DOM), 1)).astype(f32)
    pers = h * jnp.dot(onehot, gate, preferred_element_type=f32)
    t = jnp.maximum(jnp.dot(pers, T0_ref[...], preferred_element_type=f32)
                    + bt0_ref[...], 0.0)
    t = jnp.maximum(jnp.dot(t, T1_ref[...], preferred_element_type=f32)
                    + bt1_ref[...], 0.0)
    logits = jnp.sum(t * T2_ref[...], axis=-1) + bt2_ref[0, 0]
    out_ref[...] = jax.nn.sigmoid(logits)


def _full(shape):
    return pl.BlockSpec(shape, lambda i: tuple(0 for _ in shape))


_tc_forward = pl.pallas_call(
    _tc_body,
    grid=(_GRID,),
    in_specs=[
        pl.BlockSpec((_BT, _IN), lambda i: (i, 0)),
        pl.BlockSpec((_BT,), lambda i: (i,)),
        _full((1, _IN)),
        _full((1, _IN)),
        _full((_IN, 512)),
        _full((1, 512)),
        _full((512, 256)),
        _full((1, 256)),
        _full((_DOM, _D)),
        _full((_D, 256)),
        _full((1, 256)),
        _full((256, 256)),
        _full((1, 256)),
        _full((256, 128)),
        _full((1, 128)),
        _full((128, 64)),
        _full((1, 64)),
        _full((1, 64)),
        pl.BlockSpec(memory_space=pltpu.SMEM),
    ],
    out_specs=pl.BlockSpec((_BT,), lambda i: (i,)),
    out_shape=jax.ShapeDtypeStruct((_B,), jnp.float32),
    compiler_params=pltpu.CompilerParams(
        dimension_semantics=("parallel",)),
)


def kernel(x, domain_ids, shared_table, prior_table, ln_gamma, ln_beta,
           W0, b0, W1, b1, dom_emb, Wd, bd, Wg, bg,
           T0, bt0, T1, bt1, T2, bt2):
    idx = _permute_idx(x.reshape(_TOTAL).astype(jnp.int32))
    table_lin = _tr_table(shared_table.T).reshape(_V_PAD, _D)
    rows = _make_sc_gather()(table_lin, idx)
    emb = rows.reshape(_B, _IN)
    return _tc_forward(
        emb, domain_ids.astype(jnp.int32),
        ln_gamma.reshape(1, _IN), ln_beta.reshape(1, _IN),
        W0, b0.reshape(1, 512), W1, b1.reshape(1, 256),
        dom_emb, Wd, bd.reshape(1, 256), Wg, bg.reshape(1, 256),
        T0, bt0.reshape(1, 128), T1, bt1.reshape(1, 64),
        T2.reshape(1, 64), bt2.reshape(1, 1))
